# baseline (speedup 1.0000x reference)
import functools

import jax
import jax.numpy as jnp
from jax import lax
from jax.experimental import pallas as pl
from jax.experimental.pallas import tpu as pltpu
from jax.experimental.pallas import tpu_sc as plsc

B = 4
SEQ = 2048
D = 768

_info = plsc.get_sparse_core_info()
_NC = _info.num_cores
_NS = _info.num_subcores
_NW = _NC * _NS
_ROWS = SEQ // _NW

_mesh = plsc.VectorSubcoreMesh(core_axis_name="c", subcore_axis_name="s")


_HALF = _ROWS // 2


@functools.partial(
    pl.kernel,
    mesh=_mesh,
    compiler_params=pltpu.CompilerParams(skip_device_barrier=True),
    out_type=jax.ShapeDtypeStruct((B, SEQ, D), jnp.float32),
    scratch_types=[
        pltpu.VMEM((_ROWS, D), jnp.float32),
        pltpu.SemaphoreType.DMA,
        pltpu.SemaphoreType.DMA,
        pltpu.SemaphoreType.DMA,
    ],
)
def _sc_copy(tab_hbm, out_hbm, buf, rsem0, rsem1, wsem):
    wid = lax.axis_index("s") * _NC + lax.axis_index("c")
    base = wid * _ROWS
    r0 = pltpu.async_copy(
        tab_hbm.at[pl.ds(base, _HALF)], buf.at[pl.ds(0, _HALF)], rsem0
    )
    r1 = pltpu.async_copy(
        tab_hbm.at[pl.ds(base + _HALF, _HALF)], buf.at[pl.ds(_HALF, _HALF)], rsem1
    )
    writes = []
    r0.wait()
    for b in range(B):
        writes.append(
            pltpu.async_copy(
                buf.at[pl.ds(0, _HALF)], out_hbm.at[b, pl.ds(base, _HALF)], wsem
            )
        )
    r1.wait()
    for b in range(B):
        writes.append(
            pltpu.async_copy(
                buf.at[pl.ds(_HALF, _HALF)],
                out_hbm.at[b, pl.ds(base + _HALF, _HALF)],
                wsem,
            )
        )
    for w in writes:
        w.wait()


def kernel(x, pos_table):
    del x
    return _sc_copy(pos_table)


# speedup vs baseline: 1.0044x; 1.0044x over previous
import functools

import jax
import jax.numpy as jnp
from jax import lax
from jax.experimental import pallas as pl
from jax.experimental.pallas import tpu as pltpu
from jax.experimental.pallas import tpu_sc as plsc

B = 4
SEQ = 2048
D = 768

_info = plsc.get_sparse_core_info()
_NC = _info.num_cores
_NS = _info.num_subcores
_NW = _NC * _NS
_ROWS = SEQ // _NW

_mesh = plsc.VectorSubcoreMesh(core_axis_name="c", subcore_axis_name="s")


@functools.partial(
    pl.kernel,
    mesh=_mesh,
    out_type=jax.ShapeDtypeStruct((B, SEQ, D), jnp.float32),
    scratch_types=[pltpu.VMEM((_ROWS, D), jnp.float32)],
)
def _sc_copy(tab_hbm, out_hbm, buf):
    wid = lax.axis_index("s") * _NC + lax.axis_index("c")
    base = wid * _ROWS
    pltpu.sync_copy(tab_hbm.at[pl.ds(base, _ROWS)], buf)
    for b in range(B):
        pltpu.sync_copy(buf, out_hbm.at[b, pl.ds(base, _ROWS)])


def kernel(x, pos_table):
    del x
    return _sc_copy(pos_table)
